# baseline (speedup 1.0000x reference)
import functools

import jax
import jax.numpy as jnp
from jax import lax
from jax.experimental import pallas as pl
from jax.experimental.pallas import tpu as pltpu
from jax.experimental.pallas import tpu_sc as plsc

VOCAB = 100000
D = 128
B = 16384
NC, NS, L = 2, 16, 16
NW = NC * NS
BPW = B // NW
CHUNK = 128
NCHUNK = BPW // CHUNK

_mesh = plsc.VectorSubcoreMesh(
    core_axis_name="c", subcore_axis_name="s", num_cores=NC, num_subcores=NS
)


@functools.partial(
    pl.kernel,
    out_type=jax.ShapeDtypeStruct((NW, BPW, D), jnp.float32),
    mesh=_mesh,
    scratch_types=[
        pltpu.VMEM((BPW,), jnp.int32),
        pltpu.VMEM((BPW,), jnp.int32),
        pltpu.VMEM((BPW, D), jnp.float32),
        pltpu.SemaphoreType.DMA((NCHUNK,)),
        pltpu.SemaphoreType.DMA,
    ],
)
def _gather_kernel(label_hbm, table_hbm, out_hbm, lbl_v, idx_v, rows_flat, gsem, osem):
    wid = lax.axis_index("s") * NC + lax.axis_index("c")
    pltpu.sync_copy(label_hbm.at[pl.ds(wid * BPW, BPW)], lbl_v)
    for i in range(BPW // L):
        v = lbl_v[pl.ds(i * L, L)]
        idx_v[pl.ds(i * L, L)] = v * 2 + 1
    pltpu.async_copy(table_hbm.at[idx_v], rows_flat, gsem.at[0]).wait()
    pltpu.async_copy(rows_flat, out_hbm.at[wid], osem).wait()


def kernel(label, text_vectors):
    table = text_vectors.reshape(2 * VOCAB, D)
    out = _gather_kernel(label.astype(jnp.int32), table)
    return out.reshape(B, 1, D)
